# Initial kernel scaffold; baseline (speedup 1.0000x reference)
import jax
import jax.numpy as jnp
from jax.experimental import pallas as pl


def kernel(bsz, enc_w, self_w, cross_w):
    raise NotImplementedError("write your pallas kernel here")


# speedup vs baseline: 1.3178x; 1.3178x over previous
import jax
import jax.numpy as jnp
from jax.experimental import pallas as pl

L = 12
SRC = 2048 + 2
TGT = 2048 + 2
D = 1024

_ROWS = L * SRC
_RB = 600


def _copy_body(enc_ref, self_ref, cross_ref, out_ref):
    out_ref[0] = enc_ref[...]
    out_ref[1] = self_ref[...]
    out_ref[2] = cross_ref[...]


def kernel(bsz, enc_w, self_w, cross_w):
    del bsz
    enc2 = enc_w.reshape(_ROWS, D)
    self2 = self_w.reshape(_ROWS, D)
    cross2 = cross_w.reshape(_ROWS, D)
    grid = (_ROWS // _RB,)
    out = pl.pallas_call(
        _copy_body,
        grid=grid,
        in_specs=[
            pl.BlockSpec((_RB, D), lambda i: (i, 0)),
            pl.BlockSpec((_RB, D), lambda i: (i, 0)),
            pl.BlockSpec((_RB, D), lambda i: (i, 0)),
        ],
        out_specs=pl.BlockSpec((3, _RB, D), lambda i: (0, i, 0)),
        out_shape=jax.ShapeDtypeStruct((3, _ROWS, D), jnp.float32),
    )(enc2, self2, cross2)
    return out.reshape(3, L, SRC, D)


# speedup vs baseline: 1.3253x; 1.0057x over previous
import jax
import jax.numpy as jnp
from jax.experimental import pallas as pl

L = 12
SRC = 2048 + 2
TGT = 2048 + 2
D = 1024

_ROWS = L * SRC
_RB = 984


def _copy_body(enc_ref, self_ref, cross_ref, out_ref):
    out_ref[0] = enc_ref[...]
    out_ref[1] = self_ref[...]
    out_ref[2] = cross_ref[...]


def kernel(bsz, enc_w, self_w, cross_w):
    del bsz
    enc2 = enc_w.reshape(_ROWS, D)
    self2 = self_w.reshape(_ROWS, D)
    cross2 = cross_w.reshape(_ROWS, D)
    grid = (_ROWS // _RB,)
    out = pl.pallas_call(
        _copy_body,
        grid=grid,
        in_specs=[
            pl.BlockSpec((_RB, D), lambda i: (i, 0)),
            pl.BlockSpec((_RB, D), lambda i: (i, 0)),
            pl.BlockSpec((_RB, D), lambda i: (i, 0)),
        ],
        out_specs=pl.BlockSpec((3, _RB, D), lambda i: (0, i, 0)),
        out_shape=jax.ShapeDtypeStruct((3, _ROWS, D), jnp.float32),
    )(enc2, self2, cross2)
    return out.reshape(3, L, SRC, D)
